# baseline (speedup 1.0000x reference)
import functools

import jax
import jax.numpy as jnp
from jax import lax
from jax.experimental import pallas as pl
from jax.experimental.pallas import tpu as pltpu
from jax.experimental.pallas import tpu_sc as plsc

_SEQ_L = 2048


def _sal_body(adj_ref, w_ref, sal_ref):
    mean_adj = jnp.mean(adj_ref[...], axis=0)
    sal_ref[...] = jnp.dot(mean_adj, w_ref[...],
                           preferred_element_type=jnp.float32)


def _salience_t(adj_b, w):
    k_i, k_j, r_dim = adj_b.shape
    h_dim = w.shape[1]
    return pl.pallas_call(
        _sal_body,
        in_specs=[pl.BlockSpec(memory_space=pltpu.VMEM),
                  pl.BlockSpec(memory_space=pltpu.VMEM)],
        out_specs=pl.BlockSpec(memory_space=pltpu.VMEM),
        out_shape=jax.ShapeDtypeStruct((k_j, h_dim), jnp.float32),
    )(adj_b, w)


def _make_sc_row_t(rec, k, seq_l):
    zrows = seq_l // 16
    mesh = plsc.VectorSubcoreMesh(core_axis_name="c", subcore_axis_name="s")

    n_active = 8
    jpw = k // n_active

    @functools.partial(
        pl.kernel, mesh=mesh,
        out_type=jax.ShapeDtypeStruct((seq_l, rec), jnp.float32),
        scratch_types=[
            pltpu.VMEM((zrows, rec), jnp.float32),
            pltpu.VMEM((jpw, rec), jnp.float32),
            pltpu.VMEM((jpw,), jnp.int32),
        ],
        compiler_params=pltpu.CompilerParams(use_tc_tiling_on_sc=False),
    )
    def sc_row_t(sal_hbm, idx_hbm, out_hbm, zero_v, sal_v, idx_v):
        cid = lax.axis_index("c")
        sid = lax.axis_index("s")

        @pl.when(cid == 0)
        def _():
            @pl.when(sid < n_active)
            def _stage():
                pltpu.sync_copy(sal_hbm.at[pl.ds(sid * jpw, jpw)], sal_v)
                pltpu.sync_copy(idx_hbm.at[pl.ds(sid * jpw, jpw)], idx_v)

            zeros16 = jnp.zeros((16,), jnp.float32)

            def zero_step(t, _):
                for c in range(rec // 16):
                    zero_v[t, pl.ds(c * 16, 16)] = zeros16
                return 0

            lax.fori_loop(0, zrows, zero_step, 0)

            pltpu.sync_copy(zero_v, out_hbm.at[pl.ds(sid * zrows, zrows)])
            plsc.subcore_barrier()

            @pl.when(sid < n_active)
            def _scatter():
                pltpu.sync_copy(sal_v, out_hbm.at[idx_v])

    return sc_row_t


def _bcast_body(rowt_ref, out_ref, row_ref, buf_ref, sem,
                *, row_tile, seq_l, h_dim):
    row_ref[...] = rowt_ref[:, :h_dim].T
    n_tiles = seq_l // row_tile
    for h in range(h_dim):
        slot = h % 2
        if h >= 2:
            for t in range(n_tiles):
                pltpu.make_async_copy(
                    buf_ref.at[slot],
                    out_ref.at[h - 2, pl.ds(t * row_tile, row_tile)],
                    sem.at[slot]).wait()
        buf_ref[slot] = jnp.broadcast_to(row_ref[pl.ds(h, 1), :],
                                         (row_tile, seq_l))
        for t in range(n_tiles):
            pltpu.make_async_copy(
                buf_ref.at[slot],
                out_ref.at[h, pl.ds(t * row_tile, row_tile)],
                sem.at[slot]).start()

    for h in (h_dim - 2, h_dim - 1):
        slot = h % 2
        for t in range(n_tiles):
            pltpu.make_async_copy(
                buf_ref.at[slot],
                out_ref.at[h, pl.ds(t * row_tile, row_tile)],
                sem.at[slot]).wait()


def _broadcast(row_t, h_dim, seq_l, row_tile):
    return pl.pallas_call(
        functools.partial(_bcast_body, row_tile=row_tile, seq_l=seq_l,
                          h_dim=h_dim),
        in_specs=[pl.BlockSpec(memory_space=pltpu.VMEM)],
        out_specs=pl.BlockSpec(memory_space=pl.ANY),
        out_shape=jax.ShapeDtypeStruct((h_dim, seq_l, seq_l), jnp.float32),
        scratch_shapes=[
            pltpu.VMEM((h_dim, seq_l), jnp.float32),
            pltpu.VMEM((2, row_tile, seq_l), jnp.float32),
            pltpu.SemaphoreType.DMA((2,)),
        ],
    )(row_t)


def kernel(adj_matrix, top_k_indices, seq_l, relation_head_weights):
    b, k = adj_matrix.shape[:2]
    h_dim = relation_head_weights.shape[1]
    seq_l = _SEQ_L
    row_tile = 512
    rec = 16
    sc_row_t = _make_sc_row_t(rec, k, seq_l)
    w_pad = jnp.pad(relation_head_weights, ((0, 0), (0, rec - h_dim)))
    outs = []
    for bi in range(b):
        sal_t = _salience_t(adj_matrix[bi], w_pad)
        idx = top_k_indices[bi].astype(jnp.int32)
        row_t = sc_row_t(sal_t, idx)
        outs.append(_broadcast(row_t, h_dim, seq_l, row_tile))
    return jnp.stack(outs, axis=0)


# speedup vs baseline: 1.0185x; 1.0185x over previous
import functools

import jax
import jax.numpy as jnp
from jax import lax
from jax.experimental import pallas as pl
from jax.experimental.pallas import tpu as pltpu
from jax.experimental.pallas import tpu_sc as plsc

_SEQ_L = 2048


def _sal_body(adj_ref, w_ref, sal_ref):
    mean_adj = jnp.mean(adj_ref[...], axis=0)
    sal_ref[...] = jnp.dot(mean_adj, w_ref[...],
                           preferred_element_type=jnp.float32)


def _salience_t(adj_b, w):
    k_i, k_j, r_dim = adj_b.shape
    h_dim = w.shape[1]
    return pl.pallas_call(
        _sal_body,
        in_specs=[pl.BlockSpec(memory_space=pltpu.VMEM),
                  pl.BlockSpec(memory_space=pltpu.VMEM)],
        out_specs=pl.BlockSpec(memory_space=pltpu.VMEM),
        out_shape=jax.ShapeDtypeStruct((k_j, h_dim), jnp.float32),
    )(adj_b, w)


def _make_sc_row_t(rec, k, seq_l):
    zrows = seq_l // 16
    mesh = plsc.VectorSubcoreMesh(core_axis_name="c", subcore_axis_name="s")

    n_active = 8
    jpw = k // n_active

    @functools.partial(
        pl.kernel, mesh=mesh,
        out_type=jax.ShapeDtypeStruct((seq_l, rec), jnp.float32),
        scratch_types=[
            pltpu.VMEM((zrows, rec), jnp.float32),
            pltpu.VMEM((jpw, rec), jnp.float32),
            pltpu.VMEM((jpw,), jnp.int32),
        ],
    )
    def sc_row_t(sal_hbm, idx_hbm, out_hbm, zero_v, sal_v, idx_v):
        cid = lax.axis_index("c")
        sid = lax.axis_index("s")

        @pl.when(cid == 0)
        def _():
            @pl.when(sid < n_active)
            def _stage():
                pltpu.sync_copy(sal_hbm.at[pl.ds(sid * jpw, jpw)], sal_v)
                pltpu.sync_copy(idx_hbm.at[pl.ds(sid * jpw, jpw)], idx_v)

            zeros16 = jnp.zeros((16,), jnp.float32)

            def zero_step(t, _):
                for c in range(rec // 16):
                    zero_v[t, pl.ds(c * 16, 16)] = zeros16
                return 0

            lax.fori_loop(0, zrows, zero_step, 0)

            pltpu.sync_copy(zero_v, out_hbm.at[pl.ds(sid * zrows, zrows)])
            plsc.subcore_barrier()

            @pl.when(sid < n_active)
            def _scatter():
                pltpu.sync_copy(sal_v, out_hbm.at[idx_v])

    return sc_row_t


def _bcast_body(rowt_ref, out_ref, row_ref, buf_ref, sem,
                *, row_tile, seq_l, h_dim):
    row_ref[...] = rowt_ref[:, :h_dim].T
    n_tiles = seq_l // row_tile
    for h in range(h_dim):
        slot = h % 2
        if h >= 2:
            for t in range(n_tiles):
                pltpu.make_async_copy(
                    buf_ref.at[slot],
                    out_ref.at[h - 2, pl.ds(t * row_tile, row_tile)],
                    sem.at[slot]).wait()
        buf_ref[slot] = jnp.broadcast_to(row_ref[pl.ds(h, 1), :],
                                         (row_tile, seq_l))
        for t in range(n_tiles):
            pltpu.make_async_copy(
                buf_ref.at[slot],
                out_ref.at[h, pl.ds(t * row_tile, row_tile)],
                sem.at[slot]).start()

    for h in (h_dim - 2, h_dim - 1):
        slot = h % 2
        for t in range(n_tiles):
            pltpu.make_async_copy(
                buf_ref.at[slot],
                out_ref.at[h, pl.ds(t * row_tile, row_tile)],
                sem.at[slot]).wait()


def _broadcast(row_t, h_dim, seq_l, row_tile):
    return pl.pallas_call(
        functools.partial(_bcast_body, row_tile=row_tile, seq_l=seq_l,
                          h_dim=h_dim),
        in_specs=[pl.BlockSpec(memory_space=pltpu.VMEM)],
        out_specs=pl.BlockSpec(memory_space=pl.ANY),
        out_shape=jax.ShapeDtypeStruct((h_dim, seq_l, seq_l), jnp.float32),
        scratch_shapes=[
            pltpu.VMEM((h_dim, seq_l), jnp.float32),
            pltpu.VMEM((2, row_tile, seq_l), jnp.float32),
            pltpu.SemaphoreType.DMA((2,)),
        ],
    )(row_t)


def kernel(adj_matrix, top_k_indices, seq_l, relation_head_weights):
    b, k = adj_matrix.shape[:2]
    h_dim = relation_head_weights.shape[1]
    seq_l = _SEQ_L
    row_tile = 512
    rec = 128
    sc_row_t = _make_sc_row_t(rec, k, seq_l)
    w_pad = jnp.pad(relation_head_weights, ((0, 0), (0, rec - h_dim)))
    outs = []
    for bi in range(b):
        sal_t = _salience_t(adj_matrix[bi], w_pad)
        idx = top_k_indices[bi].astype(jnp.int32)
        row_t = sc_row_t(sal_t, idx)
        outs.append(_broadcast(row_t, h_dim, seq_l, row_tile))
    return jnp.stack(outs, axis=0)


# speedup vs baseline: 1.0205x; 1.0020x over previous
import functools

import jax
import jax.numpy as jnp
from jax import lax
from jax.experimental import pallas as pl
from jax.experimental.pallas import tpu as pltpu
from jax.experimental.pallas import tpu_sc as plsc

_SEQ_L = 2048


def _sal_body(adj_ref, w_ref, sal_ref):
    mean_adj = jnp.mean(adj_ref[...], axis=0)
    sal_ref[...] = jnp.dot(mean_adj, w_ref[...],
                           preferred_element_type=jnp.float32)


def _salience_t(adj_b, w):
    k_i, k_j, r_dim = adj_b.shape
    h_dim = w.shape[1]
    return pl.pallas_call(
        _sal_body,
        in_specs=[pl.BlockSpec(memory_space=pltpu.VMEM),
                  pl.BlockSpec(memory_space=pltpu.VMEM)],
        out_specs=pl.BlockSpec(memory_space=pltpu.VMEM),
        out_shape=jax.ShapeDtypeStruct((k_j, h_dim), jnp.float32),
    )(adj_b, w)


def _make_sc_row_t(rec, k, seq_l):
    zrows = seq_l // 16
    mesh = plsc.VectorSubcoreMesh(core_axis_name="c", subcore_axis_name="s")

    n_active = 8
    jpw = k // n_active

    @functools.partial(
        pl.kernel, mesh=mesh,
        out_type=jax.ShapeDtypeStruct((seq_l, rec), jnp.float32),
        scratch_types=[
            pltpu.VMEM((zrows, rec), jnp.float32),
            pltpu.VMEM((jpw, rec), jnp.float32),
            pltpu.VMEM((jpw,), jnp.int32),
        ],
    )
    def sc_row_t(sal_hbm, idx_hbm, out_hbm, zero_v, sal_v, idx_v):
        cid = lax.axis_index("c")
        sid = lax.axis_index("s")

        @pl.when(cid == 0)
        def _():
            @pl.when(sid < n_active)
            def _stage():
                pltpu.sync_copy(sal_hbm.at[pl.ds(sid * jpw, jpw)], sal_v)
                pltpu.sync_copy(idx_hbm.at[pl.ds(sid * jpw, jpw)], idx_v)

            zeros16 = jnp.zeros((16,), jnp.float32)

            def zero_step(t, _):
                for c in range(rec // 16):
                    zero_v[t, pl.ds(c * 16, 16)] = zeros16
                return 0

            lax.fori_loop(0, zrows, zero_step, 0)

            pltpu.sync_copy(zero_v, out_hbm.at[pl.ds(sid * zrows, zrows)])
            plsc.subcore_barrier()

            @pl.when(sid < n_active)
            def _scatter():
                pltpu.sync_copy(sal_v, out_hbm.at[idx_v])

    return sc_row_t


def _bcast_body(rowt_ref, out_ref, row_ref, buf_ref, sem,
                *, row_tile, seq_l, h_dim):
    row_ref[...] = rowt_ref[:, :h_dim].T
    n_tiles = seq_l // row_tile
    for h in range(h_dim):
        slot = h % 2
        if h >= 2:
            for t in range(n_tiles):
                pltpu.make_async_copy(
                    buf_ref.at[slot],
                    out_ref.at[h - 2, pl.ds(t * row_tile, row_tile)],
                    sem.at[slot]).wait()
        buf_ref[slot] = jnp.broadcast_to(row_ref[pl.ds(h, 1), :],
                                         (row_tile, seq_l))
        for t in range(n_tiles):
            pltpu.make_async_copy(
                buf_ref.at[slot],
                out_ref.at[h, pl.ds(t * row_tile, row_tile)],
                sem.at[slot]).start()

    for h in (h_dim - 2, h_dim - 1):
        slot = h % 2
        for t in range(n_tiles):
            pltpu.make_async_copy(
                buf_ref.at[slot],
                out_ref.at[h, pl.ds(t * row_tile, row_tile)],
                sem.at[slot]).wait()


def _broadcast(row_t, h_dim, seq_l, row_tile):
    return pl.pallas_call(
        functools.partial(_bcast_body, row_tile=row_tile, seq_l=seq_l,
                          h_dim=h_dim),
        in_specs=[pl.BlockSpec(memory_space=pltpu.VMEM)],
        out_specs=pl.BlockSpec(memory_space=pl.ANY),
        out_shape=jax.ShapeDtypeStruct((h_dim, seq_l, seq_l), jnp.float32),
        scratch_shapes=[
            pltpu.VMEM((h_dim, seq_l), jnp.float32),
            pltpu.VMEM((2, row_tile, seq_l), jnp.float32),
            pltpu.SemaphoreType.DMA((2,)),
        ],
    )(row_t)


def kernel(adj_matrix, top_k_indices, seq_l, relation_head_weights):
    b, k = adj_matrix.shape[:2]
    h_dim = relation_head_weights.shape[1]
    seq_l = _SEQ_L
    row_tile = 256
    rec = 128
    sc_row_t = _make_sc_row_t(rec, k, seq_l)
    w_pad = jnp.pad(relation_head_weights, ((0, 0), (0, rec - h_dim)))
    outs = []
    for bi in range(b):
        sal_t = _salience_t(adj_matrix[bi], w_pad)
        idx = top_k_indices[bi].astype(jnp.int32)
        row_t = sc_row_t(sal_t, idx)
        outs.append(_broadcast(row_t, h_dim, seq_l, row_tile))
    return jnp.stack(outs, axis=0)


# speedup vs baseline: 1.0221x; 1.0016x over previous
import functools

import jax
import jax.numpy as jnp
from jax import lax
from jax.experimental import pallas as pl
from jax.experimental.pallas import tpu as pltpu
from jax.experimental.pallas import tpu_sc as plsc

_SEQ_L = 2048


def _sal_body(adj_ref, w_ref, sal_ref):
    mean_adj = jnp.mean(adj_ref[...], axis=0)
    sal_ref[...] = jnp.dot(mean_adj, w_ref[...],
                           preferred_element_type=jnp.float32)


def _salience_t(adj_b, w):
    k_i, k_j, r_dim = adj_b.shape
    h_dim = w.shape[1]
    return pl.pallas_call(
        _sal_body,
        in_specs=[pl.BlockSpec(memory_space=pltpu.VMEM),
                  pl.BlockSpec(memory_space=pltpu.VMEM)],
        out_specs=pl.BlockSpec(memory_space=pltpu.VMEM),
        out_shape=jax.ShapeDtypeStruct((k_j, h_dim), jnp.float32),
    )(adj_b, w)


def _make_sc_row_t(rec, k, seq_l):
    zrows = seq_l // 16
    mesh = plsc.VectorSubcoreMesh(core_axis_name="c", subcore_axis_name="s")

    n_active = 8
    jpw = k // n_active

    @functools.partial(
        pl.kernel, mesh=mesh,
        out_type=jax.ShapeDtypeStruct((seq_l, rec), jnp.float32),
        scratch_types=[
            pltpu.VMEM((zrows, rec), jnp.float32),
            pltpu.VMEM((jpw, rec), jnp.float32),
            pltpu.VMEM((jpw,), jnp.int32),
        ],
    )
    def sc_row_t(sal_hbm, idx_hbm, out_hbm, zero_v, sal_v, idx_v):
        cid = lax.axis_index("c")
        sid = lax.axis_index("s")

        @pl.when(cid == 0)
        def _():
            @pl.when(sid < n_active)
            def _stage():
                pltpu.sync_copy(sal_hbm.at[pl.ds(sid * jpw, jpw)], sal_v)
                pltpu.sync_copy(idx_hbm.at[pl.ds(sid * jpw, jpw)], idx_v)

            zeros16 = jnp.zeros((16,), jnp.float32)

            def zero_step(t, _):
                for c in range(rec // 16):
                    zero_v[t, pl.ds(c * 16, 16)] = zeros16
                return 0

            lax.fori_loop(0, zrows, zero_step, 0)

            pltpu.sync_copy(zero_v, out_hbm.at[pl.ds(sid * zrows, zrows)])
            plsc.subcore_barrier()

            @pl.when(sid < n_active)
            def _scatter():
                pltpu.sync_copy(sal_v, out_hbm.at[idx_v])

    return sc_row_t


def _bcast_body(rowt_ref, out_ref, row_ref, buf_ref, sem,
                *, row_tile, seq_l, h_dim):
    row_ref[...] = rowt_ref[:, :h_dim].T
    n_tiles = seq_l // row_tile
    for h in range(h_dim):
        slot = h % 2
        if h >= 2:
            for t in range(n_tiles):
                pltpu.make_async_copy(
                    buf_ref.at[slot],
                    out_ref.at[h - 2, pl.ds(t * row_tile, row_tile)],
                    sem.at[slot]).wait()
        buf_ref[slot] = jnp.broadcast_to(row_ref[pl.ds(h, 1), :],
                                         (row_tile, seq_l))
        for t in range(n_tiles):
            pltpu.make_async_copy(
                buf_ref.at[slot],
                out_ref.at[h, pl.ds(t * row_tile, row_tile)],
                sem.at[slot]).start()

    for h in (h_dim - 2, h_dim - 1):
        slot = h % 2
        for t in range(n_tiles):
            pltpu.make_async_copy(
                buf_ref.at[slot],
                out_ref.at[h, pl.ds(t * row_tile, row_tile)],
                sem.at[slot]).wait()


def _broadcast(row_t, h_dim, seq_l, row_tile):
    return pl.pallas_call(
        functools.partial(_bcast_body, row_tile=row_tile, seq_l=seq_l,
                          h_dim=h_dim),
        in_specs=[pl.BlockSpec(memory_space=pltpu.VMEM)],
        out_specs=pl.BlockSpec(memory_space=pl.ANY),
        out_shape=jax.ShapeDtypeStruct((h_dim, seq_l, seq_l), jnp.float32),
        scratch_shapes=[
            pltpu.VMEM((h_dim, seq_l), jnp.float32),
            pltpu.VMEM((2, row_tile, seq_l), jnp.float32),
            pltpu.SemaphoreType.DMA((2,)),
        ],
    )(row_t)


def kernel(adj_matrix, top_k_indices, seq_l, relation_head_weights):
    b, k = adj_matrix.shape[:2]
    h_dim = relation_head_weights.shape[1]
    seq_l = _SEQ_L
    row_tile = 128
    rec = 128
    sc_row_t = _make_sc_row_t(rec, k, seq_l)
    w_pad = jnp.pad(relation_head_weights, ((0, 0), (0, rec - h_dim)))
    outs = []
    for bi in range(b):
        sal_t = _salience_t(adj_matrix[bi], w_pad)
        idx = top_k_indices[bi].astype(jnp.int32)
        row_t = sc_row_t(sal_t, idx)
        outs.append(_broadcast(row_t, h_dim, seq_l, row_tile))
    return jnp.stack(outs, axis=0)
